# baseline (speedup 1.0000x reference)
import jax
import jax.numpy as jnp
from jax.experimental import pallas as pl
from jax.experimental.pallas import tpu as pltpu

N = 100000
H = 128
G = 512
B = 1000
NB = N // B
NEG = -3.0e38


def _shiftR(v, d, pad):
    return jnp.concatenate(
        [jnp.full((1, d), pad, v.dtype), v[:, :-d]], axis=1)


def _shiftL(v, d, pad):
    return jnp.concatenate(
        [v[:, d:], jnp.full((1, d), pad, v.dtype)], axis=1)


def _fused(x_ref, seg_ref, wg1_ref, bg1_ref, wg2_ref,
           wpa_ref, wpb_ref, wpc_ref, bp_ref, gamma_ref, beta_ref,
           out_ref,
           sums, counts, maxs, rmax, esum, exsum):
    i = pl.program_id(0)
    x = x_ref[...]
    xb = x.astype(jnp.bfloat16)
    seg_row = seg_ref[0]

    h = jnp.maximum(
        jnp.dot(xb, wg1_ref[...], preferred_element_type=jnp.float32)
        + bg1_ref[...], 0.0)
    gate = jnp.dot(h.astype(jnp.bfloat16), wg2_ref[...],
                   preferred_element_type=jnp.float32)
    gate_row = gate.reshape(1, B)

    tail_row = seg_row != _shiftL(seg_row, 1, -1)
    bnd_row = seg_row != _shiftR(seg_row, 1, -1)
    iota_row = jax.lax.broadcasted_iota(jnp.int32, (1, B), 1)
    rs = jnp.where(bnd_row, iota_row, -1)
    d = 1
    while d < B:
        rs = jnp.maximum(rs, _shiftR(rs, d, -1))
        d *= 2

    gm = gate_row
    d = 1
    while d < B:
        ok = seg_row == _shiftR(seg_row, d, -1)
        gm = jnp.maximum(gm, jnp.where(ok, _shiftR(gm, d, NEG), NEG))
        d *= 2
    M = gm
    d = 1
    while d < B:
        okb = seg_row == _shiftL(seg_row, d, -2)
        M = jnp.maximum(M, jnp.where(okb, _shiftL(M, d, NEG), NEG))
        d *= 2
    e_row = jnp.exp(gate_row - M)

    rs_col = rs.reshape(B, 1)
    iota_col = jax.lax.broadcasted_iota(jnp.int32, (B, 1), 0)
    m = xb
    d = 1
    while d < B:
        ok_col = rs_col <= iota_col - d
        m_sh = jnp.concatenate(
            [jnp.full((d, H), NEG, jnp.bfloat16), m[:-d, :]], axis=0)
        m = jnp.maximum(m, jnp.where(ok_col, m_sh, jnp.bfloat16(NEG)))
        d *= 2

    iota_g = jax.lax.broadcasted_iota(jnp.int32, (G, 1), 0)
    oh = (iota_g == seg_row).astype(jnp.bfloat16)
    oh_tail = jnp.where(tail_row, oh, jnp.bfloat16(0))

    e_col = e_row.reshape(B, 1).astype(jnp.bfloat16)
    y = xb * e_col
    ones_col = jnp.ones((B, 1), jnp.bfloat16)
    rhs_big = jnp.concatenate([xb, y, ones_col, e_col], axis=1)
    big = jax.lax.dot_general(oh, rhs_big, (((1,), (0,)), ((), ())),
                              preferred_element_type=jnp.float32)
    s_blk = big[:, :H]
    ex_blk = big[:, H:2 * H]
    c_blk = big[:, 2 * H:2 * H + 1]
    es_blk = big[:, 2 * H + 1:2 * H + 2]

    gm_col = gm.reshape(B, 1).astype(jnp.bfloat16)
    rhs_tail = jnp.concatenate([m, gm_col, ones_col], axis=1)
    tl = jax.lax.dot_general(oh_tail, rhs_tail, (((1,), (0,)), ((), ())),
                             preferred_element_type=jnp.float32)
    present = tl[:, H + 1:H + 2] > 0
    mx_blk = jnp.where(present, tl[:, :H], NEG)
    gmx_blk = jnp.where(present, tl[:, H:H + 1], NEG)

    @pl.when(i == 0)
    def _():
        sums[...] = s_blk
        counts[...] = c_blk
        maxs[...] = mx_blk
        rmax[...] = gmx_blk
        esum[...] = es_blk
        exsum[...] = ex_blk

    @pl.when(i > 0)
    def _():
        sums[...] += s_blk
        counts[...] += c_blk
        maxs[...] = jnp.maximum(maxs[...], mx_blk)
        r_old = rmax[...]
        r_new = jnp.maximum(r_old, gmx_blk)
        scale_old = jnp.exp(r_old - r_new)
        scale_blk = jnp.exp(gmx_blk - r_new)
        esum[...] = esum[...] * scale_old + es_blk * scale_blk
        exsum[...] = exsum[...] * scale_old + ex_blk * scale_blk
        rmax[...] = r_new

    @pl.when(i == NB - 1)
    def _():
        cnt = counts[...]
        nonempty = cnt > 0
        z_mean = sums[...] / jnp.maximum(cnt, 1.0)
        z_max = jnp.where(nonempty, maxs[...], float('-inf'))
        z_attn = exsum[...] / jnp.maximum(esum[...], 1e-30)
        z = (jnp.dot(z_mean, wpa_ref[...], preferred_element_type=jnp.float32)
             + jnp.dot(z_max, wpb_ref[...], preferred_element_type=jnp.float32)
             + jnp.dot(z_attn, wpc_ref[...],
                       preferred_element_type=jnp.float32)
             + bp_ref[...])
        mu = jnp.mean(z, axis=1, keepdims=True)
        var = jnp.mean((z - mu) ** 2, axis=1, keepdims=True)
        out_ref[...] = ((z - mu) * jax.lax.rsqrt(var + 1e-5) * gamma_ref[...]
                        + beta_ref[...])


def kernel(x, batch, W_g1, b_g1, W_g2, b_g2, W_p, b_p, gamma, beta):
    seg = batch.astype(jnp.int32).reshape(NB, 1, B)
    bg1 = b_g1.reshape(1, H // 4)

    full = lambda shp: pl.BlockSpec(shp, lambda i: tuple(0 for _ in shp))
    out = pl.pallas_call(
        _fused,
        grid=(NB,),
        in_specs=[
            pl.BlockSpec((B, H), lambda i: (i, 0)),
            pl.BlockSpec((1, 1, B), lambda i: (i, 0, 0)),
            full((H, H // 4)),
            full((1, H // 4)),
            full((H // 4, 1)),
            full((H, H)), full((H, H)), full((H, H)),
            full((1, H)), full((1, H)), full((1, H)),
        ],
        out_specs=full((G, H)),
        out_shape=jax.ShapeDtypeStruct((G, H), jnp.float32),
        scratch_shapes=[
            pltpu.VMEM((G, H), jnp.float32),
            pltpu.VMEM((G, 1), jnp.float32),
            pltpu.VMEM((G, H), jnp.float32),
            pltpu.VMEM((G, 1), jnp.float32),
            pltpu.VMEM((G, 1), jnp.float32),
            pltpu.VMEM((G, H), jnp.float32),
        ],
        compiler_params=pltpu.CompilerParams(
            dimension_semantics=("arbitrary",)),
    )(x, seg,
      W_g1.astype(jnp.bfloat16), bg1, W_g2.astype(jnp.bfloat16),
      W_p[:H], W_p[H:2 * H], W_p[2 * H:], b_p.reshape(1, H),
      gamma.reshape(1, H), beta.reshape(1, H))
    return out


# speedup vs baseline: 2.3065x; 2.3065x over previous
import jax
import jax.numpy as jnp
from jax.experimental import pallas as pl
from jax.experimental.pallas import tpu as pltpu

N = 100000
H = 128
G = 512
B = 1000
NB = N // B
NEG = -3.0e38


def _shiftR(v, d, pad):
    return jnp.concatenate(
        [jnp.full((1, d), pad, v.dtype), v[:, :-d]], axis=1)


def _shiftL(v, d, pad):
    return jnp.concatenate(
        [v[:, d:], jnp.full((1, d), pad, v.dtype)], axis=1)


def _fused(x_ref, seg_ref, wg1_ref, bg1_ref, wg2_ref,
           wpa_ref, wpb_ref, wpc_ref, bp_ref, gamma_ref, beta_ref,
           out_ref,
           sums, counts, maxs, rmax, esum, exsum):
    i = pl.program_id(0)
    x = x_ref[...]
    xb = x.astype(jnp.bfloat16)
    seg_row = seg_ref[0]

    h = jnp.maximum(
        jnp.dot(xb, wg1_ref[...], preferred_element_type=jnp.float32)
        + bg1_ref[...], 0.0)
    gate = jnp.dot(h.astype(jnp.bfloat16), wg2_ref[...],
                   preferred_element_type=jnp.float32)

    tail_row = seg_row != _shiftL(seg_row, 1, -1)
    seg_col = seg_row.reshape(B, 1)

    m = xb
    gm = gate
    d = 1
    while d < B:
        seg_sh = jnp.concatenate(
            [jnp.full((d, 1), -1, jnp.int32), seg_col[:-d, :]], axis=0)
        ok_col = seg_sh == seg_col
        m_sh = jnp.concatenate(
            [jnp.full((d, H), NEG, jnp.bfloat16), m[:-d, :]], axis=0)
        m = jnp.maximum(m, jnp.where(ok_col, m_sh, jnp.bfloat16(NEG)))
        g_sh = jnp.concatenate(
            [jnp.full((d, 1), NEG, jnp.float32), gm[:-d, :]], axis=0)
        gm = jnp.maximum(gm, jnp.where(ok_col, g_sh, NEG))
        d *= 2

    iota_g = jax.lax.broadcasted_iota(jnp.int32, (G, 1), 0)
    oh = (iota_g == seg_row).astype(jnp.bfloat16)
    oh_tail = jnp.where(tail_row, oh, jnp.bfloat16(0))

    gm_col = gm.astype(jnp.bfloat16)
    ones_col = jnp.ones((B, 1), jnp.bfloat16)
    rhs_tail = jnp.concatenate([m, gm_col, ones_col], axis=1)
    tl = jax.lax.dot_general(oh_tail, rhs_tail, (((1,), (0,)), ((), ())),
                             preferred_element_type=jnp.float32)
    present = tl[:, H + 1:H + 2] > 0
    mx_blk = jnp.where(present, tl[:, :H], NEG)
    gmx_blk = jnp.where(present, tl[:, H:H + 1], NEG)

    gathered = jax.lax.dot_general(
        oh, jnp.maximum(gmx_blk, NEG).astype(jnp.bfloat16),
        (((0,), (0,)), ((), ())),
        preferred_element_type=jnp.float32)
    e = jnp.exp(gate - gathered)

    e_col = e.astype(jnp.bfloat16)
    y = xb * e_col
    rhs_big = jnp.concatenate([xb, y, ones_col, e_col], axis=1)
    big = jax.lax.dot_general(oh, rhs_big, (((1,), (0,)), ((), ())),
                              preferred_element_type=jnp.float32)
    s_blk = big[:, :H]
    ex_blk = big[:, H:2 * H]
    c_blk = big[:, 2 * H:2 * H + 1]
    es_blk = big[:, 2 * H + 1:2 * H + 2]

    @pl.when(i == 0)
    def _():
        sums[...] = s_blk
        counts[...] = c_blk
        maxs[...] = mx_blk
        rmax[...] = gmx_blk
        esum[...] = es_blk
        exsum[...] = ex_blk

    @pl.when(i > 0)
    def _():
        sums[...] += s_blk
        counts[...] += c_blk
        maxs[...] = jnp.maximum(maxs[...], mx_blk)
        r_old = rmax[...]
        r_new = jnp.maximum(r_old, gmx_blk)
        scale_old = jnp.exp(r_old - r_new)
        scale_blk = jnp.exp(gmx_blk - r_new)
        esum[...] = esum[...] * scale_old + es_blk * scale_blk
        exsum[...] = exsum[...] * scale_old + ex_blk * scale_blk
        rmax[...] = r_new

    @pl.when(i == NB - 1)
    def _():
        cnt = counts[...]
        nonempty = cnt > 0
        z_mean = sums[...] / jnp.maximum(cnt, 1.0)
        z_max = jnp.where(nonempty, maxs[...], float('-inf'))
        z_attn = exsum[...] / jnp.maximum(esum[...], 1e-30)
        z = (jnp.dot(z_mean, wpa_ref[...], preferred_element_type=jnp.float32)
             + jnp.dot(z_max, wpb_ref[...], preferred_element_type=jnp.float32)
             + jnp.dot(z_attn, wpc_ref[...],
                       preferred_element_type=jnp.float32)
             + bp_ref[...])
        mu = jnp.mean(z, axis=1, keepdims=True)
        var = jnp.mean((z - mu) ** 2, axis=1, keepdims=True)
        out_ref[...] = ((z - mu) * jax.lax.rsqrt(var + 1e-5) * gamma_ref[...]
                        + beta_ref[...])


def kernel(x, batch, W_g1, b_g1, W_g2, b_g2, W_p, b_p, gamma, beta):
    seg = batch.astype(jnp.int32).reshape(NB, 1, B)
    bg1 = b_g1.reshape(1, H // 4)

    full = lambda shp: pl.BlockSpec(shp, lambda i: tuple(0 for _ in shp))
    out = pl.pallas_call(
        _fused,
        grid=(NB,),
        in_specs=[
            pl.BlockSpec((B, H), lambda i: (i, 0)),
            pl.BlockSpec((1, 1, B), lambda i: (i, 0, 0)),
            full((H, H // 4)),
            full((1, H // 4)),
            full((H // 4, 1)),
            full((H, H)), full((H, H)), full((H, H)),
            full((1, H)), full((1, H)), full((1, H)),
        ],
        out_specs=full((G, H)),
        out_shape=jax.ShapeDtypeStruct((G, H), jnp.float32),
        scratch_shapes=[
            pltpu.VMEM((G, H), jnp.float32),
            pltpu.VMEM((G, 1), jnp.float32),
            pltpu.VMEM((G, H), jnp.float32),
            pltpu.VMEM((G, 1), jnp.float32),
            pltpu.VMEM((G, 1), jnp.float32),
            pltpu.VMEM((G, H), jnp.float32),
        ],
        compiler_params=pltpu.CompilerParams(
            dimension_semantics=("arbitrary",)),
    )(x, seg,
      W_g1.astype(jnp.bfloat16), bg1, W_g2.astype(jnp.bfloat16),
      W_p[:H], W_p[H:2 * H], W_p[2 * H:], b_p.reshape(1, H),
      gamma.reshape(1, H), beta.reshape(1, H))
    return out


# speedup vs baseline: 2.4054x; 1.0429x over previous
import jax
import jax.numpy as jnp
from jax.experimental import pallas as pl
from jax.experimental.pallas import tpu as pltpu

N = 100000
H = 128
G = 512
B = 1000
NB = N // B
NEG = -3.0e38


def _shiftR(v, d, pad):
    return jnp.concatenate(
        [jnp.full((1, d), pad, v.dtype), v[:, :-d]], axis=1)


def _shiftL(v, d, pad):
    return jnp.concatenate(
        [v[:, d:], jnp.full((1, d), pad, v.dtype)], axis=1)


def _fused(x_ref, seg_ref, wg1_ref, bg1_ref, wg2_ref,
           wpa_ref, wpb_ref, wpc_ref, bp_ref, gamma_ref, beta_ref,
           out_ref,
           sums, counts, maxs, rmax, esum, exsum):
    i = pl.program_id(0)
    x = x_ref[...]
    xb = x.astype(jnp.bfloat16)
    seg_row = seg_ref[0]

    h = jnp.maximum(
        jnp.dot(xb, wg1_ref[...], preferred_element_type=jnp.float32)
        + bg1_ref[...], 0.0)
    gate = jnp.dot(h.astype(jnp.bfloat16), wg2_ref[...],
                   preferred_element_type=jnp.float32)

    tail_row = seg_row != _shiftL(seg_row, 1, -1)
    seg_col = seg_row.reshape(B, 1)

    m = jnp.concatenate([xb, gate.astype(jnp.bfloat16)], axis=1)
    d = 1
    while d < B:
        seg_sh = jnp.concatenate(
            [jnp.full((d, 1), -1, jnp.int32), seg_col[:-d, :]], axis=0)
        ok_col = seg_sh == seg_col
        m_sh = jnp.concatenate(
            [jnp.full((d, H + 1), NEG, jnp.bfloat16), m[:-d, :]], axis=0)
        m = jnp.maximum(m, jnp.where(ok_col, m_sh, jnp.bfloat16(NEG)))
        d *= 2

    iota_g = jax.lax.broadcasted_iota(jnp.int32, (G, 1), 0)
    oh = (iota_g == seg_row).astype(jnp.bfloat16)
    oh_tail = jnp.where(tail_row, oh, jnp.bfloat16(0))

    ones_col = jnp.ones((B, 1), jnp.bfloat16)
    rhs_tail = jnp.concatenate([m, ones_col], axis=1)
    tl = jax.lax.dot_general(oh_tail, rhs_tail, (((1,), (0,)), ((), ())),
                             preferred_element_type=jnp.float32)
    present = tl[:, H + 1:H + 2] > 0
    mx_blk = jnp.where(present, tl[:, :H], NEG)
    gmx_blk = jnp.where(present, tl[:, H:H + 1], NEG)

    gathered = jax.lax.dot_general(
        oh, jnp.maximum(gmx_blk, NEG).astype(jnp.bfloat16),
        (((0,), (0,)), ((), ())),
        preferred_element_type=jnp.float32)
    e = jnp.exp(gate - gathered)

    e_col = e.astype(jnp.bfloat16)
    y = xb * e_col
    rhs_big = jnp.concatenate([xb, y, ones_col, e_col], axis=1)
    big = jax.lax.dot_general(oh, rhs_big, (((1,), (0,)), ((), ())),
                              preferred_element_type=jnp.float32)
    s_blk = big[:, :H]
    ex_blk = big[:, H:2 * H]
    c_blk = big[:, 2 * H:2 * H + 1]
    es_blk = big[:, 2 * H + 1:2 * H + 2]

    @pl.when(i == 0)
    def _():
        sums[...] = s_blk
        counts[...] = c_blk
        maxs[...] = mx_blk
        rmax[...] = gmx_blk
        esum[...] = es_blk
        exsum[...] = ex_blk

    @pl.when(i > 0)
    def _():
        sums[...] += s_blk
        counts[...] += c_blk
        maxs[...] = jnp.maximum(maxs[...], mx_blk)
        r_old = rmax[...]
        r_new = jnp.maximum(r_old, gmx_blk)
        scale_old = jnp.exp(r_old - r_new)
        scale_blk = jnp.exp(gmx_blk - r_new)
        esum[...] = esum[...] * scale_old + es_blk * scale_blk
        exsum[...] = exsum[...] * scale_old + ex_blk * scale_blk
        rmax[...] = r_new

    @pl.when(i == NB - 1)
    def _():
        cnt = counts[...]
        nonempty = cnt > 0
        z_mean = sums[...] / jnp.maximum(cnt, 1.0)
        z_max = jnp.where(nonempty, maxs[...], float('-inf'))
        z_attn = exsum[...] / jnp.maximum(esum[...], 1e-30)
        z = (jnp.dot(z_mean, wpa_ref[...], preferred_element_type=jnp.float32)
             + jnp.dot(z_max, wpb_ref[...], preferred_element_type=jnp.float32)
             + jnp.dot(z_attn, wpc_ref[...],
                       preferred_element_type=jnp.float32)
             + bp_ref[...])
        mu = jnp.mean(z, axis=1, keepdims=True)
        var = jnp.mean((z - mu) ** 2, axis=1, keepdims=True)
        out_ref[...] = ((z - mu) * jax.lax.rsqrt(var + 1e-5) * gamma_ref[...]
                        + beta_ref[...])


def kernel(x, batch, W_g1, b_g1, W_g2, b_g2, W_p, b_p, gamma, beta):
    seg = batch.astype(jnp.int32).reshape(NB, 1, B)
    bg1 = b_g1.reshape(1, H // 4)

    full = lambda shp: pl.BlockSpec(shp, lambda i: tuple(0 for _ in shp))
    out = pl.pallas_call(
        _fused,
        grid=(NB,),
        in_specs=[
            pl.BlockSpec((B, H), lambda i: (i, 0)),
            pl.BlockSpec((1, 1, B), lambda i: (i, 0, 0)),
            full((H, H // 4)),
            full((1, H // 4)),
            full((H // 4, 1)),
            full((H, H)), full((H, H)), full((H, H)),
            full((1, H)), full((1, H)), full((1, H)),
        ],
        out_specs=full((G, H)),
        out_shape=jax.ShapeDtypeStruct((G, H), jnp.float32),
        scratch_shapes=[
            pltpu.VMEM((G, H), jnp.float32),
            pltpu.VMEM((G, 1), jnp.float32),
            pltpu.VMEM((G, H), jnp.float32),
            pltpu.VMEM((G, 1), jnp.float32),
            pltpu.VMEM((G, 1), jnp.float32),
            pltpu.VMEM((G, H), jnp.float32),
        ],
        compiler_params=pltpu.CompilerParams(
            dimension_semantics=("arbitrary",)),
    )(x, seg,
      W_g1.astype(jnp.bfloat16), bg1, W_g2.astype(jnp.bfloat16),
      W_p[:H], W_p[H:2 * H], W_p[2 * H:], b_p.reshape(1, H),
      gamma.reshape(1, H), beta.reshape(1, H))
    return out
